# baseline (speedup 1.0000x reference)
import jax
import jax.numpy as jnp
import numpy as np
from jax import lax
from jax.experimental import pallas as pl
from jax.experimental.pallas import tpu as pltpu
from jax.experimental.pallas import tpu_sc as plsc

N = 10000
D = 128
E = 320000
LAYERS = 3

NC = 2
NS = 16
NW = NC * NS
LANES = 16

CHUNK = 128
CPT = 80
E_PAD = CPT * NW * CHUNK
ROWS2D = E_PAD // CHUNK

N_PAD = 10240
RPT = N_PAD // NS
NSLOT = 4
DW = D // 2


def _spmm_body(x_hbm, col_hbm, row_hbm, val_hbm, out_hbm,
               colr, rowr, valr, rows0, rows1, rowsf, acc,
               sem_g0, sem_g1, sem_s, sem_i):
    cid = lax.axis_index("c")
    sid = lax.axis_index("s")
    g = cid * NS + sid
    rows = (rows0, rows1)
    sem_g = (sem_g0, sem_g1)
    H = CHUNK // 2

    def _gather_start(j, b, s):
        pltpu.async_copy(x_hbm.at[colr.at[s, pl.ds(0, H)]],
                         rows[b].at[pl.ds(0, H)], sem_g[b])
        pltpu.async_copy(x_hbm.at[colr.at[s, pl.ds(H, H)]],
                         rows[b].at[pl.ds(H, H)], sem_g[b])

    def _gather_wait(b, s):
        pltpu.make_async_copy(x_hbm.at[colr.at[s]], rows[b],
                              sem_g[b]).wait()

    def _scatter_start(s):
        pltpu.async_copy(rowsf, acc.at[rowr.at[s]], sem_s, add=True)

    def _scatter_wait(s):
        pltpu.make_async_copy(rowsf, acc.at[rowr.at[s]], sem_s).wait()

    def _stage_start(j, s):
        base = g * CPT + j
        pltpu.async_copy(col_hbm.at[pl.ds(base, 1)],
                         colr.at[pl.ds(s, 1)], sem_i)
        pltpu.async_copy(row_hbm.at[pl.ds(base, 1)],
                         rowr.at[pl.ds(s, 1)], sem_i)
        pltpu.async_copy(val_hbm.at[pl.ds(base, 1)],
                         valr.at[pl.ds(s, 1)], sem_i)

    def _stage_wait(s):
        pltpu.make_async_copy(col_hbm.at[pl.ds(0, 1)],
                              colr.at[pl.ds(s, 1)], sem_i).wait()
        pltpu.make_async_copy(row_hbm.at[pl.ds(0, 1)],
                              rowr.at[pl.ds(s, 1)], sem_i).wait()
        pltpu.make_async_copy(val_hbm.at[pl.ds(0, 1)],
                              valr.at[pl.ds(s, 1)], sem_i).wait()

    def _scale(b, s):
        @plsc.parallel_loop(0, CHUNK // LANES, 1, unroll=4)
        def _edgeblk(eb):
            vv = valr[s, pl.ds(eb * LANES, LANES)]
            for l in range(LANES):
                ve = vv[l]
                e = eb * LANES + l
                for q in range(DW // LANES):
                    w = rows[b][e, pl.ds(q * LANES, LANES)]
                    ab = plsc.bitcast(w, jnp.bfloat16)
                    lo, hi = plsc.unpack(
                        ab, format=plsc.PackFormat.INTERLEAVED)
                    rowsf[e, pl.ds(q * 2 * LANES, LANES)] = lo * ve
                    rowsf[e, pl.ds(q * 2 * LANES + LANES, LANES)] = hi * ve

    for m in (0, 1):
        base = g * CPT + m
        pltpu.sync_copy(col_hbm.at[pl.ds(base, 1)], colr.at[pl.ds(m, 1)])
        pltpu.sync_copy(row_hbm.at[pl.ds(base, 1)], rowr.at[pl.ds(m, 1)])
        pltpu.sync_copy(val_hbm.at[pl.ds(base, 1)], valr.at[pl.ds(m, 1)])
    _stage_start(2, 2)
    _gather_start(0, 0, 0)
    _gather_start(1, 1, 1)

    z16 = jnp.zeros((LANES,), jnp.float32)

    def _zrow(r, carry):
        for q in range(D // LANES):
            rowsf[r, pl.ds(q * LANES, LANES)] = z16
        return carry
    lax.fori_loop(0, CHUNK, _zrow, 0)

    def _zero(t, carry):
        pltpu.sync_copy(rowsf,
                        acc.at[pl.ds(sid * RPT + t * CHUNK, CHUNK)])
        return carry
    lax.fori_loop(0, RPT // CHUNK, _zero, 0)
    plsc.subcore_barrier()

    def _chunk4(j, c4):
        for u in range(NSLOT):
            jj = j + u
            b = u % 2
            s = u
            _gather_wait(b, s)

            @pl.when(jj > 0)
            def _():
                _scatter_wait((s - 1) % NSLOT)
            _scale(b, s)
            _scatter_start(s)

            @pl.when(jj + 2 < CPT)
            def _():
                _stage_wait((s + 2) % NSLOT)
                _gather_start(jj + 2, b, (s + 2) % NSLOT)

            @pl.when(jj + 3 < CPT)
            def _():
                _stage_start(jj + 3, (s + 3) % NSLOT)
        return c4
    lax.fori_loop(0, CPT // NSLOT, lambda i, c: _chunk4(i * NSLOT, c), 0)
    _scatter_wait((CPT - 1) % NSLOT)
    plsc.subcore_barrier()

    pltpu.sync_copy(acc.at[pl.ds(sid * RPT, RPT)],
                    out_hbm.at[cid, pl.ds(sid * RPT, RPT)])


_spmm = pl.kernel(
    _spmm_body,
    out_type=jax.ShapeDtypeStruct((NC, N_PAD, D), jnp.float32),
    mesh=plsc.VectorSubcoreMesh(core_axis_name="c", subcore_axis_name="s"),
    compiler_params=pltpu.CompilerParams(use_tc_tiling_on_sc=False,
                                         needs_layout_passes=False),
    scratch_types=[
        pltpu.VMEM((NSLOT, CHUNK), jnp.int32),
        pltpu.VMEM((NSLOT, CHUNK), jnp.int32),
        pltpu.VMEM((NSLOT, CHUNK), jnp.float32),
        pltpu.VMEM((CHUNK, DW), jnp.int32),
        pltpu.VMEM((CHUNK, DW), jnp.int32),
        pltpu.VMEM((CHUNK, D), jnp.float32),
        pltpu.VMEM_SHARED((N_PAD, D), jnp.float32),
        pltpu.SemaphoreType.DMA,
        pltpu.SemaphoreType.DMA,
        pltpu.SemaphoreType.DMA,
        pltpu.SemaphoreType.DMA,
    ],
)


def _perm_matrix():
    p = np.zeros((D, D), np.float32)
    for i in range(D):
        q, rem = divmod(i, 32)
        t, r = divmod(rem, 16)
        p[i, 64 * t + 16 * q + r] = 1.0
    return jnp.asarray(p)


def _combine_body(p_ref, f_ref, pm_ref, xh_ref, fo_ref):
    s = p_ref[0] + p_ref[1]
    x = jnp.dot(s, pm_ref[...], preferred_element_type=jnp.float32)
    fo_ref[...] = f_ref[...] + x
    xh_ref[...] = pltpu.pack_elementwise(
        [x[:, :DW], x[:, DW:]], packed_dtype=jnp.bfloat16)


_BR = 2560


def _combine(partials, final, pm):
    return pl.pallas_call(
        _combine_body,
        grid=(N_PAD // _BR,),
        in_specs=[pl.BlockSpec((NC, _BR, D), lambda i: (0, i, 0)),
                  pl.BlockSpec((_BR, D), lambda i: (i, 0)),
                  pl.BlockSpec((D, D), lambda i: (0, 0))],
        out_specs=[pl.BlockSpec((_BR, DW), lambda i: (i, 0)),
                   pl.BlockSpec((_BR, D), lambda i: (i, 0))],
        out_shape=[jax.ShapeDtypeStruct((N_PAD, DW), jnp.int32),
                   jax.ShapeDtypeStruct((N_PAD, D), jnp.float32)],
    )(partials, final, pm)


def kernel(adj_indices, adj_values, embedding):
    row = adj_indices[0]
    col = adj_indices[1]
    pad = E_PAD - E
    zi = jnp.zeros((pad,), jnp.int32)
    colp = jnp.concatenate([col, zi]).reshape(ROWS2D, CHUNK)
    rowp = jnp.concatenate([row, zi]).reshape(ROWS2D, CHUNK)
    valp = jnp.concatenate([adj_values, jnp.zeros((pad,), jnp.float32)]
                           ).reshape(ROWS2D, CHUNK)
    x = jnp.concatenate([embedding, jnp.zeros((N_PAD - N, D), jnp.float32)])
    pm = _perm_matrix()
    final = x
    xh = lax.bitcast_convert_type(
        jnp.stack([x[:, :DW].astype(jnp.bfloat16),
                   x[:, DW:].astype(jnp.bfloat16)], axis=-1), jnp.int32)
    for _ in range(LAYERS):
        partials = _spmm(xh, colp, rowp, valp)
        xh, final = _combine(partials, final, pm)
    return final[:N]


# speedup vs baseline: 1.0338x; 1.0338x over previous
import jax
import jax.numpy as jnp
import numpy as np
from jax import lax
from jax.experimental import pallas as pl
from jax.experimental.pallas import tpu as pltpu
from jax.experimental.pallas import tpu_sc as plsc

N = 10000
D = 128
E = 320000
LAYERS = 3

NC = 2
NS = 16
NW = NC * NS
LANES = 16

CHUNK = 128
CPT = 80
E_PAD = CPT * NW * CHUNK
ROWS2D = E_PAD // CHUNK

N_PAD = 10240
RPT = N_PAD // NS
NSLOT = 4
DW = D // 2


def _spmm_body(x_hbm, col_hbm, row_hbm, val_hbm, out_hbm,
               colr, rowr, valr, rows0, rows1, rowsf, acc,
               sem_g0, sem_g1, sem_s, sem_i):
    cid = lax.axis_index("c")
    sid = lax.axis_index("s")
    g = cid * NS + sid
    rows = (rows0, rows1)
    sem_g = (sem_g0, sem_g1)
    H = CHUNK // 2

    def _gather_start(j, b, s):
        pltpu.async_copy(x_hbm.at[colr.at[s, pl.ds(0, H)]],
                         rows[b].at[pl.ds(0, H)], sem_g[b])
        pltpu.async_copy(x_hbm.at[colr.at[s, pl.ds(H, H)]],
                         rows[b].at[pl.ds(H, H)], sem_g[b])

    def _gather_wait(b, s):
        pltpu.make_async_copy(x_hbm.at[colr.at[s]], rows[b],
                              sem_g[b]).wait()

    def _scatter_start(s):
        pltpu.async_copy(rowsf, acc.at[rowr.at[s]], sem_s, add=True)

    def _scatter_wait(s):
        pltpu.make_async_copy(rowsf, acc.at[rowr.at[s]], sem_s).wait()

    def _stage_start(j, s):
        base = g * CPT + j
        pltpu.async_copy(col_hbm.at[pl.ds(base, 1)],
                         colr.at[pl.ds(s, 1)], sem_i)
        pltpu.async_copy(row_hbm.at[pl.ds(base, 1)],
                         rowr.at[pl.ds(s, 1)], sem_i)
        pltpu.async_copy(val_hbm.at[pl.ds(base, 1)],
                         valr.at[pl.ds(s, 1)], sem_i)

    def _stage_wait(s):
        pltpu.make_async_copy(col_hbm.at[pl.ds(0, 1)],
                              colr.at[pl.ds(s, 1)], sem_i).wait()
        pltpu.make_async_copy(row_hbm.at[pl.ds(0, 1)],
                              rowr.at[pl.ds(s, 1)], sem_i).wait()
        pltpu.make_async_copy(val_hbm.at[pl.ds(0, 1)],
                              valr.at[pl.ds(s, 1)], sem_i).wait()

    def _scale(b, s):
        @plsc.parallel_loop(0, CHUNK // LANES, 1, unroll=2)
        def _edgeblk(eb):
            vv = valr[s, pl.ds(eb * LANES, LANES)]
            for l in range(LANES):
                ve = vv[l]
                e = eb * LANES + l
                for q in range(DW // LANES):
                    w = rows[b][e, pl.ds(q * LANES, LANES)]
                    ab = plsc.bitcast(w, jnp.bfloat16)
                    lo, hi = plsc.unpack(
                        ab, format=plsc.PackFormat.INTERLEAVED)
                    rowsf[e, pl.ds(q * 2 * LANES, LANES)] = lo * ve
                    rowsf[e, pl.ds(q * 2 * LANES + LANES, LANES)] = hi * ve

    for m in (0, 1):
        base = g * CPT + m
        pltpu.sync_copy(col_hbm.at[pl.ds(base, 1)], colr.at[pl.ds(m, 1)])
        pltpu.sync_copy(row_hbm.at[pl.ds(base, 1)], rowr.at[pl.ds(m, 1)])
        pltpu.sync_copy(val_hbm.at[pl.ds(base, 1)], valr.at[pl.ds(m, 1)])
    _stage_start(2, 2)
    _gather_start(0, 0, 0)
    _gather_start(1, 1, 1)

    z16 = jnp.zeros((LANES,), jnp.float32)

    def _zrow(r, carry):
        for q in range(D // LANES):
            rowsf[r, pl.ds(q * LANES, LANES)] = z16
        return carry
    lax.fori_loop(0, CHUNK, _zrow, 0)

    def _zero(t, carry):
        pltpu.sync_copy(rowsf,
                        acc.at[pl.ds(sid * RPT + t * CHUNK, CHUNK)])
        return carry
    lax.fori_loop(0, RPT // CHUNK, _zero, 0)
    plsc.subcore_barrier()

    def _chunk4(j, c4):
        for u in range(NSLOT):
            jj = j + u
            b = u % 2
            s = u
            _gather_wait(b, s)

            @pl.when(jj > 0)
            def _():
                _scatter_wait((s - 1) % NSLOT)
            _scale(b, s)
            _scatter_start(s)

            @pl.when(jj + 2 < CPT)
            def _():
                _stage_wait((s + 2) % NSLOT)
                _gather_start(jj + 2, b, (s + 2) % NSLOT)

            @pl.when(jj + 3 < CPT)
            def _():
                _stage_start(jj + 3, (s + 3) % NSLOT)
        return c4
    lax.fori_loop(0, CPT // NSLOT, lambda i, c: _chunk4(i * NSLOT, c), 0)
    _scatter_wait((CPT - 1) % NSLOT)
    plsc.subcore_barrier()

    pltpu.sync_copy(acc.at[pl.ds(sid * RPT, RPT)],
                    out_hbm.at[cid, pl.ds(sid * RPT, RPT)])


_spmm = pl.kernel(
    _spmm_body,
    out_type=jax.ShapeDtypeStruct((NC, N_PAD, D), jnp.float32),
    mesh=plsc.VectorSubcoreMesh(core_axis_name="c", subcore_axis_name="s"),
    compiler_params=pltpu.CompilerParams(use_tc_tiling_on_sc=False,
                                         needs_layout_passes=False),
    scratch_types=[
        pltpu.VMEM((NSLOT, CHUNK), jnp.int32),
        pltpu.VMEM((NSLOT, CHUNK), jnp.int32),
        pltpu.VMEM((NSLOT, CHUNK), jnp.float32),
        pltpu.VMEM((CHUNK, DW), jnp.int32),
        pltpu.VMEM((CHUNK, DW), jnp.int32),
        pltpu.VMEM((CHUNK, D), jnp.float32),
        pltpu.VMEM_SHARED((N_PAD, D), jnp.float32),
        pltpu.SemaphoreType.DMA,
        pltpu.SemaphoreType.DMA,
        pltpu.SemaphoreType.DMA,
        pltpu.SemaphoreType.DMA,
    ],
)


def _perm_matrix():
    p = np.zeros((D, D), np.float32)
    for i in range(D):
        q, rem = divmod(i, 32)
        t, r = divmod(rem, 16)
        p[i, 64 * t + 16 * q + r] = 1.0
    return jnp.asarray(p)


def _combine_body(p_ref, f_ref, pm_ref, xh_ref, fo_ref):
    s = p_ref[0] + p_ref[1]
    x = jnp.dot(s, pm_ref[...], preferred_element_type=jnp.float32)
    fo_ref[...] = f_ref[...] + x
    xh_ref[...] = pltpu.pack_elementwise(
        [x[:, :DW], x[:, DW:]], packed_dtype=jnp.bfloat16)


_BR = 2560


def _combine(partials, final, pm):
    return pl.pallas_call(
        _combine_body,
        grid=(N_PAD // _BR,),
        in_specs=[pl.BlockSpec((NC, _BR, D), lambda i: (0, i, 0)),
                  pl.BlockSpec((_BR, D), lambda i: (i, 0)),
                  pl.BlockSpec((D, D), lambda i: (0, 0))],
        out_specs=[pl.BlockSpec((_BR, DW), lambda i: (i, 0)),
                   pl.BlockSpec((_BR, D), lambda i: (i, 0))],
        out_shape=[jax.ShapeDtypeStruct((N_PAD, DW), jnp.int32),
                   jax.ShapeDtypeStruct((N_PAD, D), jnp.float32)],
    )(partials, final, pm)


def kernel(adj_indices, adj_values, embedding):
    row = adj_indices[0]
    col = adj_indices[1]
    pad = E_PAD - E
    zi = jnp.zeros((pad,), jnp.int32)
    colp = jnp.concatenate([col, zi]).reshape(ROWS2D, CHUNK)
    rowp = jnp.concatenate([row, zi]).reshape(ROWS2D, CHUNK)
    valp = jnp.concatenate([adj_values, jnp.zeros((pad,), jnp.float32)]
                           ).reshape(ROWS2D, CHUNK)
    x = jnp.concatenate([embedding, jnp.zeros((N_PAD - N, D), jnp.float32)])
    pm = _perm_matrix()
    final = x
    xh = lax.bitcast_convert_type(
        jnp.stack([x[:, :DW].astype(jnp.bfloat16),
                   x[:, DW:].astype(jnp.bfloat16)], axis=-1), jnp.int32)
    for _ in range(LAYERS):
        partials = _spmm(xh, colp, rowp, valp)
        xh, final = _combine(partials, final, pm)
    return final[:N]
